# baseline (speedup 1.0000x reference)
import functools

import jax
import jax.numpy as jnp
from jax.experimental import pallas as pl
from jax.experimental.pallas import tpu as pltpu


def _pick_tile(n, candidates):
    for c in candidates:
        if n % c == 0:
            return c
    return n


def _yw_body(x_ref, w1_ref, w2_ref, y1_ref, y2_ref):
    x = x_ref[...].astype(jnp.bfloat16)
    w1 = w1_ref[...].astype(jnp.bfloat16)
    w2 = w2_ref[...].astype(jnp.bfloat16)
    dn = (((1,), (0,)), ((), ()))
    y1_ref[...] = jax.lax.dot_general(
        x, w1, dn, preferred_element_type=jnp.float32).astype(jnp.bfloat16)
    y2_ref[...] = jax.lax.dot_general(
        x, w2, dn, preferred_element_type=jnp.float32).astype(jnp.bfloat16)


def _mask_tile(a, rem_r, rem_c):
    ti, tj = a.shape
    row_ok = jax.lax.broadcasted_iota(jnp.int32, (ti, tj), 0) < rem_r
    col_ok = jax.lax.broadcasted_iota(jnp.int32, (ti, tj), 1) < rem_c
    return jnp.where(jnp.logical_and(row_ok, col_ok), a, 0.0)


def _main_body(ti, tj, n, a_ref, y1_ref, y2_ref,
               out_in_ref, out_out_ref, deg_r_ref, deg_c_ref):
    j = pl.program_id(0)
    i = pl.program_id(1)
    a = a_ref[...]
    rem_r = n - i * ti
    rem_c = n - j * tj
    a = jax.lax.cond(
        jnp.logical_and(rem_r >= ti, rem_c >= tj),
        lambda x: x,
        lambda x: _mask_tile(x, rem_r, rem_c),
        a)
    ab = a.astype(jnp.bfloat16)
    y1 = y1_ref[...]
    isl = pl.ds(i * ti, ti)
    y2 = y2_ref[isl, :]

    c_in = jax.lax.dot_general(
        ab, y1, (((1,), (0,)), ((), ())), preferred_element_type=jnp.float32)
    c_out = jax.lax.dot_general(
        ab, y2, (((0,), (0,)), ((), ())), preferred_element_type=jnp.float32)
    rs = jnp.sum(a, axis=1, keepdims=True)
    cs = jnp.sum(a, axis=0, keepdims=True).reshape(1, 1, tj)

    @pl.when(j == 0)
    def _():
        out_in_ref[isl, :] = c_in
        deg_r_ref[isl, :] = rs

    @pl.when(j > 0)
    def _():
        out_in_ref[isl, :] += c_in
        deg_r_ref[isl, :] += rs

    @pl.when(i == 0)
    def _():
        out_out_ref[...] = c_out
        deg_c_ref[...] = cs

    @pl.when(i > 0)
    def _():
        out_out_ref[...] += c_out
        deg_c_ref[...] += cs


def _epi_body(oi_ref, oo_ref, dr_ref, dc_ref, out_ref):
    r1 = 1.0 / jnp.clip(dr_ref[...], 1e-6, None)
    r2 = 1.0 / jnp.clip(dc_ref[...], 1e-6, None)
    out_ref[...] = oi_ref[...] * r1 + oo_ref[...] * r2


def kernel(X, A, W_in, W_out):
    n, d_in = X.shape
    d_out = W_in.shape[1]

    tb = _pick_tile(n, (2000, 1000, 400, 200, 80, 40, 16, 8))
    y1, y2 = pl.pallas_call(
        _yw_body,
        grid=(n // tb,),
        in_specs=[
            pl.BlockSpec((tb, d_in), lambda b: (b, 0)),
            pl.BlockSpec((d_in, d_out), lambda b: (0, 0)),
            pl.BlockSpec((d_in, d_out), lambda b: (0, 0)),
        ],
        out_specs=[
            pl.BlockSpec((tb, d_out), lambda b: (b, 0)),
            pl.BlockSpec((tb, d_out), lambda b: (b, 0)),
        ],
        out_shape=[
            jax.ShapeDtypeStruct((n, d_out), jnp.bfloat16),
            jax.ShapeDtypeStruct((n, d_out), jnp.bfloat16),
        ],
    )(X, W_in, W_out)

    ti = tj = 1024 if n >= 1024 else n
    ni = nj = -(-n // ti)
    n_pad = ni * ti
    if n_pad != n:
        pad = ((0, n_pad - n), (0, 0))
        y1 = jnp.pad(y1, pad)
        y2 = jnp.pad(y2, pad)
    out_in, out_out, deg_r, deg_c = pl.pallas_call(
        functools.partial(_main_body, ti, tj, n),
        grid=(nj, ni),
        in_specs=[
            pl.BlockSpec((ti, tj), lambda j, i: (i, j)),
            pl.BlockSpec((tj, d_out), lambda j, i: (j, 0)),
            pl.BlockSpec((n_pad, d_out), lambda j, i: (0, 0)),
        ],
        out_specs=[
            pl.BlockSpec((n_pad, d_out), lambda j, i: (0, 0)),
            pl.BlockSpec((tj, d_out), lambda j, i: (j, 0)),
            pl.BlockSpec((n_pad, 1), lambda j, i: (0, 0)),
            pl.BlockSpec((1, 1, tj), lambda j, i: (j, 0, 0)),
        ],
        out_shape=[
            jax.ShapeDtypeStruct((n_pad, d_out), jnp.float32),
            jax.ShapeDtypeStruct((n_pad, d_out), jnp.float32),
            jax.ShapeDtypeStruct((n_pad, 1), jnp.float32),
            jax.ShapeDtypeStruct((nj, 1, tj), jnp.float32),
        ],
        compiler_params=pltpu.CompilerParams(
            dimension_semantics=("arbitrary", "arbitrary"),
            vmem_limit_bytes=64 * 1024 * 1024,
        ),
    )(A, y1, y2)

    deg_c_col = deg_c.reshape(n_pad, 1)

    te = ti
    out = pl.pallas_call(
        _epi_body,
        grid=(n_pad // te,),
        in_specs=[
            pl.BlockSpec((te, d_out), lambda b: (b, 0)),
            pl.BlockSpec((te, d_out), lambda b: (b, 0)),
            pl.BlockSpec((te, 1), lambda b: (b, 0)),
            pl.BlockSpec((te, 1), lambda b: (b, 0)),
        ],
        out_specs=pl.BlockSpec((te, d_out), lambda b: (b, 0)),
        out_shape=jax.ShapeDtypeStruct((n_pad, d_out), jnp.float32),
    )(out_in, out_out, deg_r, deg_c_col)
    return out[:n]
